# baseline (speedup 1.0000x reference)
import jax
import jax.numpy as jnp
import numpy as np
from jax import lax
from jax.experimental import pallas as pl
from jax.experimental.pallas import tpu as pltpu
from jax.experimental.pallas import tpu_sc as plsc

_NW = 32
_W = 128
_TCH = 128


def _gather_body(table_hbm, ids_hbm, out_hbm, idx_v, buf0, buf1, sg, so0, so1):
    w = lax.axis_index("c") * 16 + lax.axis_index("s")
    tokens = out_hbm.shape[0]
    tok_w = tokens // _NW
    n_chunks = tok_w // _TCH
    tok_base = w * tok_w
    d_out = out_hbm.shape[1]

    pltpu.sync_copy(ids_hbm.at[w], idx_v)

    bufs = (buf0, buf1)
    sems = (so0, so1)

    @pl.loop(0, n_chunks, step=2)
    def _(g):
        for b in range(2):
            buf = bufs[b]
            so = sems[b]
            c = g + b

            @pl.when(c >= 2)
            def _():
                pltpu.make_async_copy(
                    buf.reshape(_TCH, d_out),
                    out_hbm.at[pl.ds(tok_base, _TCH), :],
                    so,
                ).wait()

            for j in range(3):
                pltpu.async_copy(
                    table_hbm.at[idx_v.at[c * 3 + j]],
                    buf.at[pl.ds(j * _W, _W), :],
                    sg,
                ).wait()

            pltpu.async_copy(
                buf.reshape(_TCH, d_out),
                out_hbm.at[pl.ds(tok_base + c * _TCH, _TCH), :],
                so,
            )

    for buf, so in ((buf0, so0), (buf1, so1)):
        pltpu.make_async_copy(
            buf.reshape(_TCH, d_out),
            out_hbm.at[pl.ds(tok_base, _TCH), :],
            so,
        ).wait()


def kernel(input_ids, r_table, g_table, b_table):
    b, t, c = input_ids.shape
    v, d = r_table.shape
    tokens = b * t
    n = tokens * c
    rows_w = n // _NW

    table = jnp.concatenate([r_table, g_table, b_table], axis=0)
    rr = np.arange(rows_w // _W)[:, None]
    ll = np.arange(_W)[None, :]
    pattern = jnp.asarray(((2 * rr + ll) % 3) * v, dtype=input_ids.dtype)
    flat_ids = input_ids.reshape(_NW, rows_w // _W, _W) + pattern

    mesh = plsc.VectorSubcoreMesh(core_axis_name="c", subcore_axis_name="s")
    gather = pl.kernel(
        _gather_body,
        out_type=jax.ShapeDtypeStruct((tokens, c * d), jnp.float32),
        mesh=mesh,
        scratch_types=[
            pltpu.VMEM((rows_w // _W, _W), jnp.int32),
            pltpu.VMEM((c * _W, d), jnp.float32),
            pltpu.VMEM((c * _W, d), jnp.float32),
            pltpu.SemaphoreType.DMA,
            pltpu.SemaphoreType.DMA,
            pltpu.SemaphoreType.DMA,
        ],
    )
    out = gather(table, flat_ids)
    return out.reshape(b, t, c * d)


# speedup vs baseline: 1.4972x; 1.4972x over previous
import jax
import jax.numpy as jnp
from jax import lax
from jax.experimental import pallas as pl
from jax.experimental.pallas import tpu as pltpu
from jax.experimental.pallas import tpu_sc as plsc

_NW = 32
_W = 128
_TCH = 128


def _gather_body(table_hbm, ids_hbm, out_hbm, idx_v, buf0, buf1, sg, so0, so1):
    w = lax.axis_index("c") * 16 + lax.axis_index("s")
    ch = ids_hbm.shape[0]
    d = table_hbm.shape[1]
    tokens = out_hbm.shape[0]
    tok_w = tokens // _NW
    n_chunks = tok_w // _TCH
    tok_base = w * tok_w

    pltpu.sync_copy(ids_hbm.at[:, w], idx_v)

    bufs = (buf0, buf1)
    sems = (so0, so1)

    def wb_dst(c, j):
        return out_hbm.at[pl.ds(tok_base + c * _TCH, _TCH), pl.ds(j * d, d)]

    @pl.loop(0, n_chunks, step=2)
    def _(g):
        for b in range(2):
            buf = bufs[b]
            so = sems[b]
            c = g + b

            @pl.when(c >= 2)
            def _():
                for j in range(ch):
                    pltpu.make_async_copy(buf.at[j], wb_dst(0, j), so).wait()

            for j in range(ch):
                pltpu.async_copy(
                    table_hbm.at[idx_v.at[j, c]], buf.at[j], sg
                ).wait()

            for j in range(ch):
                pltpu.async_copy(buf.at[j], wb_dst(c, j), so)

    for buf, so in ((buf0, so0), (buf1, so1)):
        for j in range(ch):
            pltpu.make_async_copy(buf.at[j], wb_dst(0, j), so).wait()


def kernel(input_ids, r_table, g_table, b_table):
    b, t, c = input_ids.shape
    v, d = r_table.shape
    tokens = b * t

    table = jnp.concatenate([r_table, g_table, b_table], axis=0)
    offsets = jnp.arange(c, dtype=input_ids.dtype) * v
    ids3 = lax.reshape(
        input_ids, (c, _NW, tokens // _W // _NW, _W), dimensions=(2, 0, 1)
    )
    ids3 = ids3 + offsets[:, None, None, None]

    mesh = plsc.VectorSubcoreMesh(core_axis_name="c", subcore_axis_name="s")
    gather = pl.kernel(
        _gather_body,
        out_type=jax.ShapeDtypeStruct((tokens, c * d), jnp.float32),
        mesh=mesh,
        scratch_types=[
            pltpu.VMEM((c, tokens // _W // _NW, _W), jnp.int32),
            pltpu.VMEM((c, _TCH, d), jnp.float32),
            pltpu.VMEM((c, _TCH, d), jnp.float32),
            pltpu.SemaphoreType.DMA,
            pltpu.SemaphoreType.DMA,
            pltpu.SemaphoreType.DMA,
        ],
    )
    out = gather(table, ids3)
    return out.reshape(b, t, c * d)


# speedup vs baseline: 2.1162x; 1.4135x over previous
import jax
import jax.numpy as jnp
from jax import lax
from jax.experimental import pallas as pl
from jax.experimental.pallas import tpu as pltpu
from jax.experimental.pallas import tpu_sc as plsc

_NW = 32
_W = 128
_TCH = 128


def _gather_body(
    table_hbm, ids_hbm, out_hbm, table_sh, idx_v, buf0, buf1, sg, so0, so1
):
    s = lax.axis_index("s")
    w = lax.axis_index("c") * 16 + s
    ch = ids_hbm.shape[0]
    d = table_hbm.shape[1]
    tokens = out_hbm.shape[0]
    tok_w = tokens // _NW
    n_chunks = tok_w // _TCH
    tok_base = w * tok_w

    n_sh = table_sh.shape[0]

    @pl.when(s == 0)
    def _():
        pltpu.sync_copy(table_hbm.at[pl.ds(0, n_sh), :], table_sh)

    pltpu.sync_copy(ids_hbm.at[:, w], idx_v)
    plsc.subcore_barrier()

    bufs = (buf0, buf1)
    sems = (so0, so1)

    def wb_dst(c, j):
        return out_hbm.at[pl.ds(tok_base + c * _TCH, _TCH), pl.ds(j * d, d)]

    @pl.loop(0, n_chunks, step=2)
    def _(g):
        for b in range(2):
            buf = bufs[b]
            so = sems[b]
            c = g + b

            @pl.when(c >= 2)
            def _():
                for j in range(ch):
                    pltpu.make_async_copy(buf.at[j], wb_dst(0, j), so).wait()

            for j in range(ch):
                src = table_sh if j < 2 else table_hbm
                pltpu.async_copy(
                    src.at[idx_v.at[j, c]], buf.at[j], sg
                ).wait()

            for j in range(ch):
                pltpu.async_copy(buf.at[j], wb_dst(c, j), so)

    for buf, so in ((buf0, so0), (buf1, so1)):
        for j in range(ch):
            pltpu.make_async_copy(buf.at[j], wb_dst(0, j), so).wait()


def kernel(input_ids, r_table, g_table, b_table):
    b, t, c = input_ids.shape
    v, d = r_table.shape
    tokens = b * t

    table = jnp.concatenate([r_table, g_table, b_table], axis=0)
    offsets = jnp.arange(c, dtype=input_ids.dtype) * v
    ids3 = lax.reshape(
        input_ids, (c, _NW, tokens // _W // _NW, _W), dimensions=(2, 0, 1)
    )
    ids3 = ids3 + offsets[:, None, None, None]

    mesh = plsc.VectorSubcoreMesh(core_axis_name="c", subcore_axis_name="s")
    gather = pl.kernel(
        _gather_body,
        out_type=jax.ShapeDtypeStruct((tokens, c * d), jnp.float32),
        mesh=mesh,
        scratch_types=[
            pltpu.VMEM_SHARED((2 * v, d), jnp.float32),
            pltpu.VMEM((c, tokens // _W // _NW, _W), jnp.int32),
            pltpu.VMEM((c, _TCH, d), jnp.float32),
            pltpu.VMEM((c, _TCH, d), jnp.float32),
            pltpu.SemaphoreType.DMA,
            pltpu.SemaphoreType.DMA,
            pltpu.SemaphoreType.DMA,
        ],
    )
    out = gather(table, ids3)
    return out.reshape(b, t, c * d)


# speedup vs baseline: 3.1938x; 1.5092x over previous
import jax
import jax.numpy as jnp
from jax import lax
from jax.experimental import pallas as pl
from jax.experimental.pallas import tpu as pltpu
from jax.experimental.pallas import tpu_sc as plsc

_NW = 32
_W = 128
_TCH = 128


def _gather_body(
    table_hbm, ids_hbm, out_hbm, table_sh, idx_v, buf0, buf1, sg, so0, so1
):
    s = lax.axis_index("s")
    w = lax.axis_index("c") * 16 + s
    ch = ids_hbm.shape[0]
    d = table_hbm.shape[1]
    tokens = out_hbm.shape[0]
    tok_w = tokens // _NW
    n_chunks = tok_w // _TCH
    tok_base = w * tok_w

    n_sh = table_sh.shape[0]

    @pl.when(s == 0)
    def _():
        pltpu.sync_copy(table_hbm.at[pl.ds(0, n_sh), :], table_sh)

    plsc.subcore_barrier()

    bufs = (buf0, buf1)
    sems = (so0, so1)
    half = idx_v.shape[1]

    def wb_dst(c, j):
        return out_hbm.at[pl.ds(tok_base + c * _TCH, _TCH), pl.ds(j * d, d)]

    for h in range(n_chunks // half):
        pltpu.sync_copy(ids_hbm.at[:, w, pl.ds(h * half, half), :], idx_v)

        @pl.loop(0, half, step=2)
        def _(g):
            for b in range(2):
                buf = bufs[b]
                so = sems[b]
                c = h * half + g + b

                @pl.when(c >= 2)
                def _():
                    for j in range(ch):
                        pltpu.make_async_copy(
                            buf.at[j], wb_dst(0, j), so
                        ).wait()

                for j in range(ch):
                    pltpu.async_copy(
                        table_sh.at[idx_v.at[j, g + b]], buf.at[j], sg
                    ).wait()

                for j in range(ch):
                    pltpu.async_copy(buf.at[j], wb_dst(c, j), so)

    for buf, so in ((buf0, so0), (buf1, so1)):
        for j in range(ch):
            pltpu.make_async_copy(buf.at[j], wb_dst(0, j), so).wait()


def kernel(input_ids, r_table, g_table, b_table):
    b, t, c = input_ids.shape
    v, d = r_table.shape
    tokens = b * t

    table = jnp.concatenate([r_table, g_table, b_table], axis=0)
    offsets = jnp.arange(c, dtype=input_ids.dtype) * v
    ids3 = lax.reshape(
        input_ids, (c, _NW, tokens // _W // _NW, _W), dimensions=(2, 0, 1)
    )
    ids3 = ids3 + offsets[:, None, None, None]

    mesh = plsc.VectorSubcoreMesh(core_axis_name="c", subcore_axis_name="s")
    gather = pl.kernel(
        _gather_body,
        out_type=jax.ShapeDtypeStruct((tokens, c * d), jnp.float32),
        mesh=mesh,
        compiler_params=pltpu.CompilerParams(
            internal_scratch_in_bytes=128 * 1024,
        ),
        scratch_types=[
            pltpu.VMEM_SHARED((c * v, d), jnp.float32),
            pltpu.VMEM((c, tokens // _W // _NW // 2, _W), jnp.int32),
            pltpu.VMEM((c, _TCH, d), jnp.float32),
            pltpu.VMEM((c, _TCH, d), jnp.float32),
            pltpu.SemaphoreType.DMA,
            pltpu.SemaphoreType.DMA,
            pltpu.SemaphoreType.DMA,
        ],
    )
    out = gather(table, ids3)
    return out.reshape(b, t, c * d)


# speedup vs baseline: 3.3380x; 1.0451x over previous
import jax
import jax.numpy as jnp
from jax import lax
from jax.experimental import pallas as pl
from jax.experimental.pallas import tpu as pltpu
from jax.experimental.pallas import tpu_sc as plsc

_NW = 32
_W = 128
_TCH = 128


def _gather_body(
    table_hbm, ids_hbm, out_hbm, table_sh, idx_v, buf0, buf1, sg, so0, so1
):
    s = lax.axis_index("s")
    w = lax.axis_index("c") * 16 + s
    ch = ids_hbm.shape[0]
    d = table_hbm.shape[1]
    tokens = out_hbm.shape[0]
    tok_w = tokens // _NW
    n_chunks = tok_w // _TCH
    tok_base = w * tok_w

    n_sh = table_sh.shape[0]

    @pl.when(s == 0)
    def _():
        pltpu.sync_copy(table_hbm.at[pl.ds(0, n_sh), :], table_sh)

    plsc.subcore_barrier()

    bufs = (buf0, buf1)
    sems = (so0, so1)
    half = idx_v.shape[1]

    def wb_dst(c, j):
        return out_hbm.at[pl.ds(tok_base + c * _TCH, _TCH), pl.ds(j * d, d)]

    for h in range(n_chunks // half):
        pltpu.sync_copy(ids_hbm.at[:, w, pl.ds(h * half, half), :], idx_v)

        @pl.loop(0, half, step=2)
        def _(g):
            for b in range(2):
                buf = bufs[b]
                so = sems[b]
                c = h * half + g + b

                @pl.when(c >= 2)
                def _():
                    for j in range(ch):
                        pltpu.make_async_copy(
                            buf.at[j], wb_dst(0, j), so
                        ).wait()

                for j in range(ch):
                    pltpu.async_copy(
                        table_sh.at[idx_v.at[j, g + b]], buf.at[j], sg
                    )
                for j in range(ch):
                    pltpu.make_async_copy(
                        table_sh.at[idx_v.at[j, g + b]], buf.at[j], sg
                    ).wait()

                for j in range(ch):
                    pltpu.async_copy(buf.at[j], wb_dst(c, j), so)

    for buf, so in ((buf0, so0), (buf1, so1)):
        for j in range(ch):
            pltpu.make_async_copy(buf.at[j], wb_dst(0, j), so).wait()


def kernel(input_ids, r_table, g_table, b_table):
    b, t, c = input_ids.shape
    v, d = r_table.shape
    tokens = b * t

    table = jnp.concatenate([r_table, g_table, b_table], axis=0)
    offsets = jnp.arange(c, dtype=input_ids.dtype) * v
    ids3 = lax.reshape(
        input_ids, (c, _NW, tokens // _W // _NW, _W), dimensions=(2, 0, 1)
    )
    ids3 = ids3 + offsets[:, None, None, None]

    mesh = plsc.VectorSubcoreMesh(core_axis_name="c", subcore_axis_name="s")
    gather = pl.kernel(
        _gather_body,
        out_type=jax.ShapeDtypeStruct((tokens, c * d), jnp.float32),
        mesh=mesh,
        compiler_params=pltpu.CompilerParams(
            internal_scratch_in_bytes=128 * 1024,
        ),
        scratch_types=[
            pltpu.VMEM_SHARED((c * v, d), jnp.float32),
            pltpu.VMEM((c, tokens // _W // _NW // 2, _W), jnp.int32),
            pltpu.VMEM((c, _TCH, d), jnp.float32),
            pltpu.VMEM((c, _TCH, d), jnp.float32),
            pltpu.SemaphoreType.DMA,
            pltpu.SemaphoreType.DMA,
            pltpu.SemaphoreType.DMA,
        ],
    )
    out = gather(table, ids3)
    return out.reshape(b, t, c * d)
